# baseline (speedup 1.0000x reference)
import functools

import jax
import jax.numpy as jnp
from jax import lax
from jax.experimental import pallas as pl
from jax.experimental.pallas import tpu as pltpu
from jax.experimental.pallas import tpu_sc as plsc

_LANES = 16
_KB = 2048


@functools.lru_cache(maxsize=None)
def _build_scatter(V, T):
    info = plsc.get_sparse_core_info()
    NC, NS = info.num_cores, info.num_subcores
    NW = NC * NS
    TPW = T // NW
    Vp = ((V + NS * 128 - 1) // (NS * 128)) * NS * 128
    CS = Vp // NS
    assert CS % 128 == 0

    mesh = plsc.VectorSubcoreMesh(core_axis_name="c", subcore_axis_name="s")

    @functools.partial(
        pl.kernel,
        out_type=jax.ShapeDtypeStruct((NC, Vp), jnp.float32),
        mesh=mesh,
        compiler_params=pltpu.CompilerParams(needs_layout_passes=False),
        scratch_types=[
            pltpu.VMEM((TPW,), jnp.int32),
            pltpu.VMEM((TPW,), jnp.float32),
            pltpu.VMEM((CS,), jnp.float32),
            pltpu.VMEM_SHARED((Vp,), jnp.float32),
        ],
    )
    def scatter_kernel(idx_hbm, coeff_hbm, out_hbm, idx_w, coeff_w, zer_v, s_sh):
        cid = lax.axis_index("c")
        sid = lax.axis_index("s")
        row = sid * NC + cid
        pltpu.sync_copy(idx_hbm.at[row], idx_w)
        pltpu.sync_copy(coeff_hbm.at[row], coeff_w)

        z = jnp.zeros((_LANES,), jnp.float32)

        def zero(i, carry):
            zer_v[pl.ds(i * _LANES, _LANES)] = z
            return carry

        lax.fori_loop(0, CS // _LANES, zero, None)
        pltpu.sync_copy(zer_v, s_sh.at[pl.ds(sid * CS, CS)])
        plsc.subcore_barrier()
        pltpu.sync_copy(coeff_w, s_sh.at[idx_w], add=True)
        plsc.subcore_barrier()
        pltpu.sync_copy(s_sh.at[pl.ds(sid * CS, CS)],
                        out_hbm.at[cid].at[pl.ds(sid * CS, CS)])

    return scatter_kernel


@functools.lru_cache(maxsize=None)
def _build_matvec(B, V, NC, Vp):
    grid = Vp // _KB
    tail_valid = V - (grid - 1) * _KB
    NG = _KB // 128

    def accum(xb, sb, acc_ref):
        t = xb[:, 0:128] * sb[:, 0:128]
        for g in range(1, NG):
            t = t + xb[:, g * 128:(g + 1) * 128] * sb[:, g * 128:(g + 1) * 128]
        acc_ref[...] += t

    def body(x_ref, s_ref, rhs_ref, out_ref, acc_ref):
        k = pl.program_id(0)
        sb = s_ref[0:1, :] + s_ref[1:2, :]

        @pl.when(k == 0)
        def _():
            acc_ref[...] = jnp.zeros_like(acc_ref)

        @pl.when(k < grid - 1)
        def _():
            accum(x_ref[...], sb, acc_ref)

        @pl.when(k == grid - 1)
        def _():
            lane = lax.broadcasted_iota(jnp.int32, (B, _KB), 1)
            xb = jnp.where(lane < tail_valid, x_ref[...], 0.0)
            accum(xb, sb, acc_ref)
            lhs = jnp.sum(acc_ref[...], axis=1)
            out_ref[...] = (lhs <= rhs_ref[0]).astype(jnp.int32)

    return pl.pallas_call(
        body,
        grid=(grid,),
        out_shape=jax.ShapeDtypeStruct((B,), jnp.int32),
        in_specs=[
            pl.BlockSpec((B, _KB), lambda k: (0, k)),
            pl.BlockSpec((NC, _KB), lambda k: (0, k)),
            pl.BlockSpec(memory_space=pltpu.SMEM),
        ],
        out_specs=pl.BlockSpec((B,), lambda k: (0,)),
        scratch_shapes=[pltpu.VMEM((B, 128), jnp.float32)],
        compiler_params=pltpu.CompilerParams(
            dimension_semantics=("arbitrary",)),
    )


def kernel(x, coeff_tensor, indices_tensor, rhs):
    B, V = x.shape
    T = indices_tensor.shape[0]
    info = plsc.get_sparse_core_info()
    NW = info.num_cores * info.num_subcores
    idx2 = indices_tensor.reshape(NW, T // NW)
    coeff2 = coeff_tensor.reshape(NW, T // NW)
    s = _build_scatter(V, T)(idx2, coeff2)
    rhs_arr = jnp.full((1,), rhs, dtype=jnp.float32)
    out = _build_matvec(B, V, s.shape[0], s.shape[1])(x, s, rhs_arr)
    return out.astype(bool)
